# baseline (speedup 1.0000x reference)
import jax
import jax.numpy as jnp
from jax import lax
from jax.experimental import pallas as pl
from jax.experimental.pallas import tpu as pltpu
from jax.experimental.pallas import tpu_sc as plsc

ROWS = 16 * 4 * 2 * 14
COLS = 4096
NSC = 3276
LANES = 16
NUM_CORES = 2
NUM_SUBCORES = 16
NW = NUM_CORES * NUM_SUBCORES
NOUT = ROWS * NSC

WIN = 204 * 128
NWIN = 224
KPER = NWIN // NW
RLOAD = 9
MAIN = NWIN * WIN
MINI = (NOUT - MAIN) // 8
MINI_R0 = MAIN // NSC
MINI_J0 = MAIN - MINI_R0 * NSC


def _wrap(j, r, step):
    j = j + step
    w = j >= NSC
    return jnp.where(w, j - NSC, j), r + w.astype(jnp.int32)


def _body(x_hbm, idx_hbm, out_hbm, idx_v, row_v, stage_v):
    wid = lax.axis_index("s") * NUM_CORES + lax.axis_index("c")
    lane = lax.iota(jnp.int32, LANES)

    pltpu.sync_copy(idx_hbm, idx_v)

    def window(f, r0, j0, nrows, ngrp):
        load0 = jnp.minimum(r0, jnp.int32(ROWS - nrows))
        pltpu.sync_copy(
            x_hbm.at[pl.ds(load0 * COLS, nrows * COLS)],
            row_v.at[pl.ds(0, nrows * COLS)],
        )
        rb0 = (r0 - load0) * COLS

        def grp(g, carry):
            jrel, rbase = carry
            jv = jrel + lane
            m = jv >= NSC
            jv = jnp.where(m, jv - NSC, jv)
            iv = plsc.load_gather(idx_v, [jv])
            addr = jnp.where(m, rbase + COLS, rbase) + iv
            stage_v[pl.ds(g * LANES, LANES)] = plsc.load_gather(row_v, [addr])
            jn = jrel + LANES
            w = jn >= NSC
            return jnp.where(w, jn - NSC, jn), jnp.where(
                w, rbase + COLS, rbase
            )

        plsc.parallel_loop(0, ngrp, 1, unroll=4, carry=(j0, rb0))(grp)
        pltpu.sync_copy(stage_v.at[pl.ds(0, ngrp * LANES)], out_hbm.at[pl.ds(f, ngrp * LANES)])

    def adv(i, c):
        j, r = _wrap(c[0], c[1], jnp.int32(WIN - 7 * NSC))
        return j, r + 7

    j0, r0 = lax.fori_loop(0, wid, adv, (jnp.int32(0), jnp.int32(0)))

    for k in range(KPER):
        f = (wid + NW * k) * WIN
        window(f, r0, j0, RLOAD, WIN // LANES)
        j0, r0 = _wrap(j0, r0, jnp.int32(NW * WIN - 255 * NSC))
        r0 = r0 + 255

    @pl.when(wid < 8)
    def _():
        def madv(i, c):
            return _wrap(c[0], c[1], jnp.int32(MINI))

        jm, rm = lax.fori_loop(
            0, wid, madv, (jnp.int32(MINI_J0), jnp.int32(MINI_R0))
        )
        window(MAIN + wid * MINI, rm, jm, 2, MINI // LANES)


_gather = pl.kernel(
    _body,
    out_type=jax.ShapeDtypeStruct((NOUT,), jnp.float32),
    mesh=plsc.VectorSubcoreMesh(core_axis_name="c", subcore_axis_name="s"),
    scratch_types=[
        pltpu.VMEM((NSC,), jnp.int32),
        pltpu.VMEM((RLOAD * COLS,), jnp.float32),
        pltpu.VMEM((WIN,), jnp.float32),
    ],
    compiler_params=pltpu.CompilerParams(needs_layout_passes=False),
)


@jax.jit
def kernel(inputs, sc_ind):
    x = inputs.reshape(-1)
    idx = sc_ind.astype(jnp.int32)
    return _gather(x, idx).reshape(16, 4, 2, 14, NSC)


# speedup vs baseline: 1.0368x; 1.0368x over previous
import jax
import jax.numpy as jnp
from jax import lax
from jax.experimental import pallas as pl
from jax.experimental.pallas import tpu as pltpu
from jax.experimental.pallas import tpu_sc as plsc

ROWS = 16 * 4 * 2 * 14
COLS = 4096
NSC = 3276
LANES = 16
NGRP = (NSC + LANES - 1) // LANES
IDX_PAD = NGRP * LANES
NUM_CORES = 2
NUM_SUBCORES = 16
NW = NUM_CORES * NUM_SUBCORES
NOUT = ROWS * NSC

WIN = 204 * 128
NWIN = 224
KPER = NWIN // NW
RLOAD = 9
MAIN = NWIN * WIN
MINI = (NOUT - MAIN) // 8
MINI_R0 = MAIN // NSC
MINI_J0 = MAIN - MINI_R0 * NSC


def _wrap(j, r, step):
    j = j + step
    w = j >= NSC
    return jnp.where(w, j - NSC, j), r + w.astype(jnp.int32)


def _body(x_hbm, idx_hbm, out_hbm, idx_v, row_v, stage_v):
    wid = lax.axis_index("s") * NUM_CORES + lax.axis_index("c")
    lane = lax.iota(jnp.int32, LANES)

    pltpu.sync_copy(idx_hbm, idx_v)

    def main_window(f, r0, j0):
        pltpu.sync_copy(x_hbm.at[pl.ds(r0 * COLS, RLOAD * COLS)], row_v)

        def grp(g):
            o = g * LANES
            iv = idx_v[pl.ds(o, LANES)]
            ol = o + lane
            olj = ol - j0
            live = ol < NSC
            for t in range(1, 7):
                vals = plsc.load_gather(row_v, [iv + jnp.int32(t * COLS)])
                plsc.store_scatter(
                    stage_v, [olj + jnp.int32(t * NSC)], vals, mask=live
                )

        plsc.parallel_loop(0, NGRP, 1, unroll=2)(grp)

        def bgrp(g):
            o = g * LANES
            iv = idx_v[pl.ds(o, LANES)]
            ol = o + lane
            olj = ol - j0
            live = ol < NSC
            v0 = plsc.load_gather(row_v, [iv])
            plsc.store_scatter(
                stage_v, [olj], v0, mask=live & (ol >= j0)
            )
            v7 = plsc.load_gather(row_v, [iv + jnp.int32(7 * COLS)])
            p7 = olj + jnp.int32(7 * NSC)
            plsc.store_scatter(
                stage_v, [p7], v7, mask=live & (p7 < WIN)
            )

        plsc.parallel_loop(0, NGRP, 1, unroll=4)(bgrp)

        def hgrp(g):
            o = g * LANES
            iv = idx_v[pl.ds(o, LANES)]
            ol = o + lane
            p8 = (ol - j0) + jnp.int32(8 * NSC)
            vals = plsc.load_gather(row_v, [iv + jnp.int32(8 * COLS)])
            plsc.store_scatter(stage_v, [p8], vals, mask=p8 < WIN)

        g_hi = jnp.maximum(j0 - (8 * NSC - WIN), 0) + (LANES - 1)
        plsc.parallel_loop(0, g_hi // LANES, 1, unroll=4)(hgrp)

        pltpu.sync_copy(stage_v, out_hbm.at[pl.ds(f, WIN)])

    def adv(i, c):
        j, r = _wrap(c[0], c[1], jnp.int32(WIN - 7 * NSC))
        return j, r + 7

    j0, r0 = lax.fori_loop(0, wid, adv, (jnp.int32(0), jnp.int32(0)))

    for k in range(KPER):
        f = (wid + NW * k) * WIN
        main_window(f, r0, j0)
        j0, r0 = _wrap(j0, r0, jnp.int32(NW * WIN - 255 * NSC))
        r0 = r0 + 255

    @pl.when(wid < 8)
    def _():
        def madv(i, c):
            return _wrap(c[0], c[1], jnp.int32(MINI))

        jm, rm = lax.fori_loop(
            0, wid, madv, (jnp.int32(MINI_J0), jnp.int32(MINI_R0))
        )
        load0 = jnp.minimum(rm, jnp.int32(ROWS - 2))
        pltpu.sync_copy(
            x_hbm.at[pl.ds(load0 * COLS, 2 * COLS)],
            row_v.at[pl.ds(0, 2 * COLS)],
        )
        rb = (rm - load0) * COLS

        def mgrp(g):
            o = g * LANES
            iv = idx_v[pl.ds(o, LANES)]
            ol = o + lane
            olj = ol - jm
            live = ol < NSC
            v0 = plsc.load_gather(row_v, [iv + rb])
            plsc.store_scatter(
                stage_v,
                [olj],
                v0,
                mask=live & (ol >= jm) & (olj < MINI),
            )
            p1 = olj + jnp.int32(NSC)
            v1 = plsc.load_gather(row_v, [iv + (rb + jnp.int32(COLS))])
            plsc.store_scatter(stage_v, [p1], v1, mask=live & (p1 < MINI))

        plsc.parallel_loop(0, NGRP, 1, unroll=4)(mgrp)
        pltpu.sync_copy(
            stage_v.at[pl.ds(0, MINI)],
            out_hbm.at[pl.ds(MAIN + wid * MINI, MINI)],
        )


_gather = pl.kernel(
    _body,
    out_type=jax.ShapeDtypeStruct((NOUT,), jnp.float32),
    mesh=plsc.VectorSubcoreMesh(core_axis_name="c", subcore_axis_name="s"),
    scratch_types=[
        pltpu.VMEM((IDX_PAD,), jnp.int32),
        pltpu.VMEM((RLOAD * COLS,), jnp.float32),
        pltpu.VMEM((WIN,), jnp.float32),
    ],
    compiler_params=pltpu.CompilerParams(needs_layout_passes=False),
)


@jax.jit
def kernel(inputs, sc_ind):
    x = inputs.reshape(-1)
    idx = jnp.concatenate(
        [sc_ind.astype(jnp.int32), jnp.zeros((IDX_PAD - NSC,), jnp.int32)]
    )
    return _gather(x, idx).reshape(16, 4, 2, 14, NSC)


# speedup vs baseline: 2.1284x; 2.0528x over previous
import jax
import jax.numpy as jnp
from jax import lax
from jax.experimental import pallas as pl
from jax.experimental.pallas import tpu as pltpu
from jax.experimental.pallas import tpu_sc as plsc

ROWS = 14
COLS = 4096
NSC = 3276
LANES = 16
NGRP = (NSC + LANES - 1) // LANES
IDX_PAD = NGRP * LANES
NUM_CORES = 2
NUM_SUBCORES = 16
NW = NUM_CORES * NUM_SUBCORES
NSLAB = 128
SPT = NSLAB // NW


def _body(x_hbm, idx_hbm, out_hbm, idx_v, row_v, stage_v):
    wid = lax.axis_index("s") * NUM_CORES + lax.axis_index("c")
    lane = lax.iota(jnp.int32, LANES)

    pltpu.sync_copy(idx_hbm, idx_v)

    for k in range(SPT):
        slab = wid * SPT + k
        pltpu.sync_copy(x_hbm.at[slab], row_v)

        def grp(g):
            o = g * LANES
            iv = idx_v[pl.ds(o, LANES)]
            live = (o + lane) < NSC
            for r in range(ROWS):
                rs = jnp.full((LANES,), r, jnp.int32)
                vals = plsc.load_gather(row_v, [rs, iv])
                plsc.store_scatter(
                    stage_v, [rs, o + lane], vals, mask=live
                )

        plsc.parallel_loop(0, NGRP, 1, unroll=1)(grp)
        pltpu.sync_copy(stage_v, out_hbm.at[slab])


_gather = pl.kernel(
    _body,
    out_type=jax.ShapeDtypeStruct((NSLAB, ROWS, NSC), jnp.float32),
    mesh=plsc.VectorSubcoreMesh(core_axis_name="c", subcore_axis_name="s"),
    scratch_types=[
        pltpu.VMEM((IDX_PAD,), jnp.int32),
        pltpu.VMEM((ROWS, COLS), jnp.float32),
        pltpu.VMEM((ROWS, NSC), jnp.float32),
    ],
    compiler_params=pltpu.CompilerParams(needs_layout_passes=False),
)


@jax.jit
def kernel(inputs, sc_ind):
    x = inputs.reshape(NSLAB, ROWS, COLS)
    idx = jnp.concatenate(
        [sc_ind.astype(jnp.int32), jnp.zeros((IDX_PAD - NSC,), jnp.int32)]
    )
    return _gather(x, idx).reshape(16, 4, 2, 14, NSC)
